# baseline (speedup 1.0000x reference)
import functools

import jax
import jax.numpy as jnp
from jax import lax
from jax.experimental import pallas as pl
from jax.experimental.pallas import tpu as pltpu
from jax.experimental.pallas import tpu_sc as plsc

D = 6
F = 64
ROWW = 16
SPW = 8
CHUNK = 1024
SUB = 128
NC = 2
NS = 16
NW = NC * NS
BN = 2048
PC = 128
STG = 512


def _table_tc_kernel(xt_ref, wt_ref, s_ref, o_ref):
    parts = []
    for c in range(BN // PC):
        parts.append(jnp.dot(xt_ref[:, c * 6 * PC:(c + 1) * 6 * PC],
                             s_ref[...], preferred_element_type=jnp.float32))
    xm = jnp.concatenate(parts, axis=1)
    tproj = jnp.dot(wt_ref[...], xm, preferred_element_type=jnp.float32)
    q = jnp.sum(xm * xm, axis=0, keepdims=True) * (1.0 / 128.0)
    rowid = lax.broadcasted_iota(jnp.int32, tproj.shape, 0)
    o_ref[...] = jnp.where(rowid == 7, q, tproj)


def _build_table_t(xt, wt, s, v):
    grid = (v + BN - 1) // BN
    return pl.pallas_call(
        _table_tc_kernel,
        grid=(grid,),
        in_specs=[
            pl.BlockSpec((F, 6 * BN), lambda i: (0, i)),
            pl.BlockSpec((ROWW, F), lambda i: (0, 0)),
            pl.BlockSpec((6 * PC, PC), lambda i: (0, 0)),
        ],
        out_specs=pl.BlockSpec((ROWW, BN), lambda i: (0, i)),
        out_shape=jax.ShapeDtypeStruct((ROWW, v), jnp.float32),
    )(xt, wt, s)


def _stage_transpose(tt_hbm, shared, tmp, tbuf, sid, v, share, lane):
    off = jnp.minimum(sid * share, v - share)
    n_st = (share + STG - 1) // STG
    last = share - STG

    def st_body(ci, carry):
        start = off + jnp.minimum(ci * STG, last)
        pltpu.sync_copy(tt_hbm.at[:, pl.ds(start, STG)], tmp)

        def g_body(g, c):
            rid = lane + g * 16
            for j in range(8):
                vv = tmp[j, pl.ds(g * 16, 16)]
                plsc.store_scatter(tbuf, [rid, jnp.full((16,), j, jnp.int32)],
                                   vv)
            return c

        lax.fori_loop(0, STG // 16, g_body, 0)
        pltpu.sync_copy(tbuf, shared.at[pl.ds(start, STG)])
        return carry

    lax.fori_loop(0, n_st, st_body, 0)


def _sc_body(ttx_hbm, tte_hbm, row_hbm, col_hbm, c_hbm, out_hbm,
             ridx0, ridx1, cidx0, cidx1, xr0, xr1, er0, er1, ob0, ob1, cv,
             tmp, tbuf, txs, tes,
             s_i0, s_i1, s_g0, s_g1, s_o0, s_o1):
    n_inc = out_hbm.shape[1]
    per_w = n_inc // NW
    n_chunks = 2 * ((per_w + 2 * CHUNK - 1) // (2 * CHUNK))
    last_start = per_w - CHUNK

    ridx = [ridx0, ridx1]
    cidx = [cidx0, cidx1]
    xr = [xr0, xr1]
    er = [er0, er1]
    ob = [ob0, ob1]
    s_i = [s_i0, s_i1]
    s_g = [s_g0, s_g1]
    s_o = [s_o0, s_o1]

    cid = lax.axis_index("c")
    sid = lax.axis_index("s")
    w = sid * NC + cid
    base_w = w * per_w

    pltpu.sync_copy(c_hbm, cv)
    lane = lax.iota(jnp.int32, 16)

    _stage_transpose(ttx_hbm, txs, tmp, tbuf, sid, ttx_hbm.shape[1], 3128,
                     lane)
    _stage_transpose(tte_hbm, tes, tmp, tbuf, sid, tte_hbm.shape[1], 1568,
                     lane)
    plsc.subcore_barrier()

    cjs = [cv[j, :] for j in range(6)]

    def chunk_base(c):
        return base_w + jnp.minimum(c * CHUNK, last_start)

    def idx_copies(buf, c):
        base = chunk_base(c)
        return [pltpu.make_async_copy(row_hbm.at[pl.ds(base, CHUNK)],
                                      ridx[buf], s_i[buf]),
                pltpu.make_async_copy(col_hbm.at[pl.ds(base, CHUNK)],
                                      cidx[buf], s_i[buf])]

    def gather_copies(buf):
        cps = []
        for j in range(CHUNK // SUB):
            sl = pl.ds(j * SUB, SUB)
            cps.append(pltpu.make_async_copy(
                txs.at[ridx[buf].at[sl]], xr[buf].at[sl], s_g[buf]))
            cps.append(pltpu.make_async_copy(
                tes.at[cidx[buf].at[sl]], er[buf].at[sl], s_g[buf]))
        return cps

    def out_copy(buf, c):
        return pltpu.make_async_copy(
            ob[buf], out_hbm.at[:, pl.ds(chunk_base(c), CHUNK)], s_o[buf])

    def compute(buf, c):
        xrb, erb, obb = xr[buf], er[buf], ob[buf]

        def group_body(gi, inner):
            rid = lane + gi * 16
            a = [plsc.load_gather(xrb, [rid, jnp.full((16,), k, jnp.int32)])
                 for k in range(8)]
            b = [plsc.load_gather(erb, [rid, jnp.full((16,), k, jnp.int32)])
                 for k in range(8)]
            mu = a[6] + b[6]
            var = (a[7] + b[7]) - mu * mu + 1e-5
            vi = plsc.bitcast(var, jnp.int32)
            y = plsc.bitcast(jnp.int32(0x5F3759DF) - (vi >> 1), jnp.float32)
            for _ in range(3):
                y = y * (1.5 - 0.5 * var * y * y)
            for jo in range(6):
                z = (a[jo] + b[jo]) * y + cjs[jo]
                s = 1.0 / (1.0 + jnp.exp(-z))
                obb[jo, pl.ds(gi * 16, 16)] = s
            return inner

        lax.fori_loop(0, CHUNK // 16, group_body, 0)

    for cp in idx_copies(0, 0):
        cp.start()
    for cp in idx_copies(1, 1):
        cp.start()
    for cp in idx_copies(0, 0):
        cp.wait()
    for cp in gather_copies(0):
        cp.start()

    def iter_body(i, carry):
        for buf in (0, 1):
            c = 2 * i + buf
            other = 1 - buf

            @pl.when(c + 1 < n_chunks)
            def _():
                for cp in idx_copies(other, c + 1):
                    cp.wait()
                for cp in gather_copies(other):
                    cp.start()

            for cp in gather_copies(buf):
                cp.wait()

            @pl.when(c + 2 < n_chunks)
            def _():
                for cp in idx_copies(buf, c + 2):
                    cp.start()

            @pl.when(c >= 2)
            def _():
                out_copy(buf, c - 2).wait()

            compute(buf, c)
            out_copy(buf, c).start()
        return carry

    lax.fori_loop(0, n_chunks // 2, iter_body, 0)
    out_copy(0, n_chunks - 2).wait()
    out_copy(1, n_chunks - 1).wait()


def kernel(x, e, hyperedge_index, ln_gamma, ln_beta, W, b):
    num_nodes = x.shape[0] // D
    num_edges = e.shape[0] // D
    n_inc = hyperedge_index.shape[1]

    row = jnp.asarray(hyperedge_index[0], jnp.int32)
    col = jnp.asarray(hyperedge_index[1], jnp.int32)

    wg = ln_gamma[:, None] * W
    s = jnp.sum(wg, axis=0)
    cvec = ln_beta @ W + b
    wx = wg[:F] - s[None, :] * (1.0 / 128.0)
    we = wg[F:] - s[None, :] * (1.0 / 128.0)
    cpad = jnp.broadcast_to(cvec[:, None], (6, 16)).astype(jnp.float32)

    def _wt(wside):
        wt = jnp.zeros((ROWW, F), jnp.float32)
        wt = wt.at[:6, :].set(wside.T)
        wt = wt.at[6, :].set(1.0 / 128.0)
        return wt

    pool = jnp.kron(jnp.eye(PC, dtype=jnp.float32),
                    jnp.full((D, 1), 1.0 / D, jnp.float32))

    ttx = _build_table_t(x.T, _wt(wx), pool, num_nodes)
    tte = _build_table_t(e.T, _wt(we), pool, num_edges)

    mesh = plsc.VectorSubcoreMesh(core_axis_name="c", subcore_axis_name="s")
    sc_fn = functools.partial(
        pl.kernel,
        out_type=jax.ShapeDtypeStruct((6, n_inc), jnp.float32),
        mesh=mesh,
        scratch_types=(
            [pltpu.VMEM((CHUNK,), jnp.int32)] * 4 +
            [pltpu.VMEM((CHUNK, SPW), jnp.float32)] * 4 +
            [pltpu.VMEM((6, CHUNK), jnp.float32)] * 2 +
            [pltpu.VMEM((6, 16), jnp.float32)] +
            [pltpu.VMEM((ROWW, STG), jnp.float32)] +
            [pltpu.VMEM((STG, SPW), jnp.float32)] +
            [pltpu.VMEM_SHARED((num_nodes, SPW), jnp.float32)] +
            [pltpu.VMEM_SHARED((num_edges, SPW), jnp.float32)] +
            [pltpu.SemaphoreType.DMA] * 6
        ),
        compiler_params=pltpu.CompilerParams(
            needs_layout_passes=False, use_tc_tiling_on_sc=False),
    )(_sc_body)
    return sc_fn(ttx, tte, row, col, cpad).T


# speedup vs baseline: 1.2473x; 1.2473x over previous
import functools

import jax
import jax.numpy as jnp
from jax import lax
from jax.experimental import pallas as pl
from jax.experimental.pallas import tpu as pltpu
from jax.experimental.pallas import tpu_sc as plsc

D = 6
F = 64
ROWW = 16
SPW = 8
CHUNK = 1024
SUB = 128
NC = 2
NS = 16
NW = NC * NS
BN = 2048
PC = 128
STG = 512


def _table_tc_kernel(xt_ref, wt_ref, s_ref, o_ref):
    parts = []
    for c in range(BN // PC):
        parts.append(jnp.dot(xt_ref[:, c * 6 * PC:(c + 1) * 6 * PC],
                             s_ref[...], preferred_element_type=jnp.float32))
    xm = jnp.concatenate(parts, axis=1)
    tproj = jnp.dot(wt_ref[...], xm, preferred_element_type=jnp.float32)
    q = jnp.sum(xm * xm, axis=0, keepdims=True) * (1.0 / 128.0)
    rowid = lax.broadcasted_iota(jnp.int32, tproj.shape, 0)
    o_ref[...] = jnp.where(rowid == 7, q, tproj)


def _build_table_t(xt, wt, s, v):
    grid = (v + BN - 1) // BN
    return pl.pallas_call(
        _table_tc_kernel,
        grid=(grid,),
        in_specs=[
            pl.BlockSpec((F, 6 * BN), lambda i: (0, i)),
            pl.BlockSpec((ROWW, F), lambda i: (0, 0)),
            pl.BlockSpec((6 * PC, PC), lambda i: (0, 0)),
        ],
        out_specs=pl.BlockSpec((ROWW, BN), lambda i: (0, i)),
        out_shape=jax.ShapeDtypeStruct((ROWW, v), jnp.float32),
    )(xt, wt, s)


def _stage_transpose(tt_hbm, shared, tmp, tbuf, sid, v, share, lane):
    off = jnp.minimum(sid * share, v - share)
    n_st = (share + STG - 1) // STG
    last = share - STG

    def st_body(ci, carry):
        start = off + jnp.minimum(ci * STG, last)
        pltpu.sync_copy(tt_hbm.at[:, pl.ds(start, STG)], tmp)

        def g_body(g, c):
            rid = lane + g * 16
            for j in range(8):
                vv = tmp[j, pl.ds(g * 16, 16)]
                plsc.store_scatter(tbuf, [rid, jnp.full((16,), j, jnp.int32)],
                                   vv)
            return c

        lax.fori_loop(0, STG // 16, g_body, 0)
        pltpu.sync_copy(tbuf, shared.at[pl.ds(start, STG)])
        return carry

    lax.fori_loop(0, n_st, st_body, 0)


def _sc_body(ttx_hbm, tte_hbm, row_hbm, col_hbm, c_hbm, out_hbm,
             ridx0, ridx1, cidx0, cidx1, xr0, xr1, er0, er1, ob0, ob1, cv,
             tmp, tbuf, txs, tes,
             s_i0, s_i1, s_g0, s_g1, s_o0, s_o1):
    n_inc = out_hbm.shape[0]
    per_w = n_inc // NW
    n_chunks = 2 * ((per_w + 2 * CHUNK - 1) // (2 * CHUNK))
    last_start = per_w - CHUNK

    ridx = [ridx0, ridx1]
    cidx = [cidx0, cidx1]
    xr = [xr0, xr1]
    er = [er0, er1]
    ob = [ob0, ob1]
    s_i = [s_i0, s_i1]
    s_g = [s_g0, s_g1]
    s_o = [s_o0, s_o1]

    cid = lax.axis_index("c")
    sid = lax.axis_index("s")
    w = sid * NC + cid
    base_w = w * per_w

    pltpu.sync_copy(c_hbm, cv)
    lane = lax.iota(jnp.int32, 16)

    _stage_transpose(ttx_hbm, txs, tmp, tbuf, sid, ttx_hbm.shape[1], 3128,
                     lane)
    _stage_transpose(tte_hbm, tes, tmp, tbuf, sid, tte_hbm.shape[1], 1568,
                     lane)
    plsc.subcore_barrier()

    cjs = [cv[j, :] for j in range(6)]

    def chunk_base(c):
        return base_w + jnp.minimum(c * CHUNK, last_start)

    def idx_copies(buf, c):
        base = chunk_base(c)
        return [pltpu.make_async_copy(row_hbm.at[pl.ds(base, CHUNK)],
                                      ridx[buf], s_i[buf]),
                pltpu.make_async_copy(col_hbm.at[pl.ds(base, CHUNK)],
                                      cidx[buf], s_i[buf])]

    def gather_copies(buf):
        cps = []
        for j in range(CHUNK // SUB):
            sl = pl.ds(j * SUB, SUB)
            cps.append(pltpu.make_async_copy(
                txs.at[ridx[buf].at[sl]], xr[buf].at[sl], s_g[buf]))
            cps.append(pltpu.make_async_copy(
                tes.at[cidx[buf].at[sl]], er[buf].at[sl], s_g[buf]))
        return cps

    def out_copy(buf, c):
        return pltpu.make_async_copy(
            ob[buf], out_hbm.at[pl.ds(chunk_base(c), CHUNK)], s_o[buf])

    def compute(buf, c):
        xrb, erb, obb = xr[buf], er[buf], ob[buf]

        def group_body(gi, inner):
            rid = lane + gi * 16
            a = [plsc.load_gather(xrb, [rid, jnp.full((16,), k, jnp.int32)])
                 for k in range(8)]
            b = [plsc.load_gather(erb, [rid, jnp.full((16,), k, jnp.int32)])
                 for k in range(8)]
            mu = a[6] + b[6]
            var = (a[7] + b[7]) - mu * mu + 1e-5
            vi = plsc.bitcast(var, jnp.int32)
            y = plsc.bitcast(jnp.int32(0x5F3759DF) - (vi >> 1), jnp.float32)
            for _ in range(3):
                y = y * (1.5 - 0.5 * var * y * y)
            for jo in range(6):
                z = (a[jo] + b[jo]) * y + cjs[jo]
                s = 1.0 / (1.0 + jnp.exp(-z))
                plsc.store_scatter(obb, [rid, jnp.full((16,), jo, jnp.int32)],
                                   s)
            return inner

        lax.fori_loop(0, CHUNK // 16, group_body, 0)

    for cp in idx_copies(0, 0):
        cp.start()
    for cp in idx_copies(1, 1):
        cp.start()
    for cp in idx_copies(0, 0):
        cp.wait()
    for cp in gather_copies(0):
        cp.start()

    def iter_body(i, carry):
        for buf in (0, 1):
            c = 2 * i + buf
            other = 1 - buf

            @pl.when(c + 1 < n_chunks)
            def _():
                for cp in idx_copies(other, c + 1):
                    cp.wait()
                for cp in gather_copies(other):
                    cp.start()

            for cp in gather_copies(buf):
                cp.wait()

            @pl.when(c + 2 < n_chunks)
            def _():
                for cp in idx_copies(buf, c + 2):
                    cp.start()

            @pl.when(c >= 2)
            def _():
                out_copy(buf, c - 2).wait()

            compute(buf, c)
            out_copy(buf, c).start()
        return carry

    lax.fori_loop(0, n_chunks // 2, iter_body, 0)
    out_copy(0, n_chunks - 2).wait()
    out_copy(1, n_chunks - 1).wait()


def kernel(x, e, hyperedge_index, ln_gamma, ln_beta, W, b):
    num_nodes = x.shape[0] // D
    num_edges = e.shape[0] // D
    n_inc = hyperedge_index.shape[1]

    row = jnp.asarray(hyperedge_index[0], jnp.int32)
    col = jnp.asarray(hyperedge_index[1], jnp.int32)

    wg = ln_gamma[:, None] * W
    s = jnp.sum(wg, axis=0)
    cvec = ln_beta @ W + b
    wx = wg[:F] - s[None, :] * (1.0 / 128.0)
    we = wg[F:] - s[None, :] * (1.0 / 128.0)
    cpad = jnp.broadcast_to(cvec[:, None], (6, 16)).astype(jnp.float32)

    def _wt(wside):
        wt = jnp.zeros((ROWW, F), jnp.float32)
        wt = wt.at[:6, :].set(wside.T)
        wt = wt.at[6, :].set(1.0 / 128.0)
        return wt

    pool = jnp.kron(jnp.eye(PC, dtype=jnp.float32),
                    jnp.full((D, 1), 1.0 / D, jnp.float32))

    ttx = _build_table_t(x.T, _wt(wx), pool, num_nodes)
    tte = _build_table_t(e.T, _wt(we), pool, num_edges)

    mesh = plsc.VectorSubcoreMesh(core_axis_name="c", subcore_axis_name="s")
    sc_fn = functools.partial(
        pl.kernel,
        out_type=jax.ShapeDtypeStruct((n_inc, 6), jnp.float32),
        mesh=mesh,
        scratch_types=(
            [pltpu.VMEM((CHUNK,), jnp.int32)] * 4 +
            [pltpu.VMEM((CHUNK, SPW), jnp.float32)] * 4 +
            [pltpu.VMEM((CHUNK, 6), jnp.float32)] * 2 +
            [pltpu.VMEM((6, 16), jnp.float32)] +
            [pltpu.VMEM((ROWW, STG), jnp.float32)] +
            [pltpu.VMEM((STG, SPW), jnp.float32)] +
            [pltpu.VMEM_SHARED((num_nodes, SPW), jnp.float32)] +
            [pltpu.VMEM_SHARED((num_edges, SPW), jnp.float32)] +
            [pltpu.SemaphoreType.DMA] * 6
        ),
        compiler_params=pltpu.CompilerParams(
            needs_layout_passes=False, use_tc_tiling_on_sc=False),
    )(_sc_body)
    return sc_fn(ttx, tte, row, col, cpad)


# speedup vs baseline: 1.3024x; 1.0442x over previous
import functools

import jax
import jax.numpy as jnp
from jax import lax
from jax.experimental import pallas as pl
from jax.experimental.pallas import tpu as pltpu
from jax.experimental.pallas import tpu_sc as plsc

D = 6
F = 64
ROWW = 16
SPW = 8
CHUNK = 1024
SUB = 128
NC = 2
NS = 16
NW = NC * NS
BN = 2048
PC = 128
STG = 512


def _table_tc_kernel(xt_ref, wt_ref, s_ref, o_ref):
    parts = []
    for c in range(BN // PC):
        parts.append(jnp.dot(xt_ref[:, c * 6 * PC:(c + 1) * 6 * PC],
                             s_ref[...], preferred_element_type=jnp.float32))
    xm = jnp.concatenate(parts, axis=1)
    tproj = jnp.dot(wt_ref[...], xm, preferred_element_type=jnp.float32)
    q = jnp.sum(xm * xm, axis=0, keepdims=True) * (1.0 / 128.0)
    rowid = lax.broadcasted_iota(jnp.int32, tproj.shape, 0)
    o_ref[...] = jnp.where(rowid == 7, q, tproj)


def _build_table_t(xt, wt, s, v):
    grid = (v + BN - 1) // BN
    return pl.pallas_call(
        _table_tc_kernel,
        grid=(grid,),
        in_specs=[
            pl.BlockSpec((F, 6 * BN), lambda i: (0, i)),
            pl.BlockSpec((ROWW, F), lambda i: (0, 0)),
            pl.BlockSpec((6 * PC, PC), lambda i: (0, 0)),
        ],
        out_specs=pl.BlockSpec((ROWW, BN), lambda i: (0, i)),
        out_shape=jax.ShapeDtypeStruct((ROWW, v), jnp.float32),
    )(xt, wt, s)


def _stage_transpose(tt_hbm, shared, tmp, tbuf, sid, v, share, lane):
    off = jnp.minimum(sid * share, v - share)
    n_st = (share + STG - 1) // STG
    last = share - STG

    def st_body(ci, carry):
        start = off + jnp.minimum(ci * STG, last)
        pltpu.sync_copy(tt_hbm.at[:, pl.ds(start, STG)], tmp)

        def g_body(g, c):
            rid = lane + g * 16
            for j in range(8):
                vv = tmp[j, pl.ds(g * 16, 16)]
                plsc.store_scatter(tbuf, [rid, jnp.full((16,), j, jnp.int32)],
                                   vv)
            return c

        lax.fori_loop(0, STG // 16, g_body, 0)
        pltpu.sync_copy(tbuf, shared.at[pl.ds(start, STG)])
        return carry

    lax.fori_loop(0, n_st, st_body, 0)


def _sc_body(ttx_hbm, tte_hbm, he_hbm, c_hbm, out_hbm,
             ridx0, ridx1, cidx0, cidx1, xr0, xr1, er0, er1, ob0, ob1, cv,
             tmp, tbuf, txs, tes,
             s_i0, s_i1, s_g0, s_g1, s_o0, s_o1):
    n_inc = out_hbm.shape[0]
    per_w = n_inc // NW
    n_chunks = 2 * ((per_w + 2 * CHUNK - 1) // (2 * CHUNK))
    last_start = per_w - CHUNK

    ridx = [ridx0, ridx1]
    cidx = [cidx0, cidx1]
    xr = [xr0, xr1]
    er = [er0, er1]
    ob = [ob0, ob1]
    s_i = [s_i0, s_i1]
    s_g = [s_g0, s_g1]
    s_o = [s_o0, s_o1]

    cid = lax.axis_index("c")
    sid = lax.axis_index("s")
    w = sid * NC + cid
    base_w = w * per_w

    pltpu.sync_copy(c_hbm, cv)
    lane = lax.iota(jnp.int32, 16)

    _stage_transpose(ttx_hbm, txs, tmp, tbuf, sid, ttx_hbm.shape[1], 3128,
                     lane)
    _stage_transpose(tte_hbm, tes, tmp, tbuf, sid, tte_hbm.shape[1], 1568,
                     lane)
    plsc.subcore_barrier()

    cjs = [cv[j, :] for j in range(6)]

    def chunk_base(c):
        return base_w + jnp.minimum(c * CHUNK, last_start)

    def idx_copies(buf, c):
        base = chunk_base(c)
        return [pltpu.make_async_copy(he_hbm.at[0, pl.ds(base, CHUNK)],
                                      ridx[buf], s_i[buf]),
                pltpu.make_async_copy(he_hbm.at[1, pl.ds(base, CHUNK)],
                                      cidx[buf], s_i[buf])]

    def gather_copies(buf):
        cps = []
        for j in range(CHUNK // SUB):
            sl = pl.ds(j * SUB, SUB)
            cps.append(pltpu.make_async_copy(
                txs.at[ridx[buf].at[sl]], xr[buf].at[sl], s_g[buf]))
            cps.append(pltpu.make_async_copy(
                tes.at[cidx[buf].at[sl]], er[buf].at[sl], s_g[buf]))
        return cps

    def out_copy(buf, c):
        return pltpu.make_async_copy(
            ob[buf], out_hbm.at[pl.ds(chunk_base(c), CHUNK)], s_o[buf])

    def compute(buf, c):
        xrb, erb, obb = xr[buf], er[buf], ob[buf]

        def group_body(gi, inner):
            rid = lane + gi * 16
            a = [plsc.load_gather(xrb, [rid, jnp.full((16,), k, jnp.int32)])
                 for k in range(8)]
            b = [plsc.load_gather(erb, [rid, jnp.full((16,), k, jnp.int32)])
                 for k in range(8)]
            mu = a[6] + b[6]
            var = (a[7] + b[7]) - mu * mu + 1e-5
            vi = plsc.bitcast(var, jnp.int32)
            y = plsc.bitcast(jnp.int32(0x5F3759DF) - (vi >> 1), jnp.float32)
            for _ in range(3):
                y = y * (1.5 - 0.5 * var * y * y)
            for jo in range(6):
                z = (a[jo] + b[jo]) * y + cjs[jo]
                s = 1.0 / (1.0 + jnp.exp(-z))
                plsc.store_scatter(obb, [rid, jnp.full((16,), jo, jnp.int32)],
                                   s)
            return inner

        lax.fori_loop(0, CHUNK // 16, group_body, 0)

    for cp in idx_copies(0, 0):
        cp.start()
    for cp in idx_copies(1, 1):
        cp.start()
    for cp in idx_copies(0, 0):
        cp.wait()
    for cp in gather_copies(0):
        cp.start()

    def iter_body(i, carry):
        for buf in (0, 1):
            c = 2 * i + buf
            other = 1 - buf

            @pl.when(c + 1 < n_chunks)
            def _():
                for cp in idx_copies(other, c + 1):
                    cp.wait()
                for cp in gather_copies(other):
                    cp.start()

            for cp in gather_copies(buf):
                cp.wait()

            @pl.when(c + 2 < n_chunks)
            def _():
                for cp in idx_copies(buf, c + 2):
                    cp.start()

            @pl.when(c >= 2)
            def _():
                out_copy(buf, c - 2).wait()

            compute(buf, c)
            out_copy(buf, c).start()
        return carry

    lax.fori_loop(0, n_chunks // 2, iter_body, 0)
    out_copy(0, n_chunks - 2).wait()
    out_copy(1, n_chunks - 1).wait()


def kernel(x, e, hyperedge_index, ln_gamma, ln_beta, W, b):
    num_nodes = x.shape[0] // D
    num_edges = e.shape[0] // D
    n_inc = hyperedge_index.shape[1]

    wg = ln_gamma[:, None] * W
    s = jnp.sum(wg, axis=0)
    cvec = ln_beta @ W + b
    wx = wg[:F] - s[None, :] * (1.0 / 128.0)
    we = wg[F:] - s[None, :] * (1.0 / 128.0)
    cpad = jnp.broadcast_to(cvec[:, None], (6, 16)).astype(jnp.float32)

    def _wt(wside):
        wt = jnp.zeros((ROWW, F), jnp.float32)
        wt = wt.at[:6, :].set(wside.T)
        wt = wt.at[6, :].set(1.0 / 128.0)
        return wt

    pool = jnp.kron(jnp.eye(PC, dtype=jnp.float32),
                    jnp.full((D, 1), 1.0 / D, jnp.float32))

    ttx = _build_table_t(x.T, _wt(wx), pool, num_nodes)
    tte = _build_table_t(e.T, _wt(we), pool, num_edges)

    mesh = plsc.VectorSubcoreMesh(core_axis_name="c", subcore_axis_name="s")
    sc_fn = functools.partial(
        pl.kernel,
        out_type=jax.ShapeDtypeStruct((n_inc, 6), jnp.float32),
        mesh=mesh,
        scratch_types=(
            [pltpu.VMEM((CHUNK,), jnp.int32)] * 4 +
            [pltpu.VMEM((CHUNK, SPW), jnp.float32)] * 4 +
            [pltpu.VMEM((CHUNK, 6), jnp.float32)] * 2 +
            [pltpu.VMEM((6, 16), jnp.float32)] +
            [pltpu.VMEM((ROWW, STG), jnp.float32)] +
            [pltpu.VMEM((STG, SPW), jnp.float32)] +
            [pltpu.VMEM_SHARED((num_nodes, SPW), jnp.float32)] +
            [pltpu.VMEM_SHARED((num_edges, SPW), jnp.float32)] +
            [pltpu.SemaphoreType.DMA] * 6
        ),
        compiler_params=pltpu.CompilerParams(
            needs_layout_passes=False, use_tc_tiling_on_sc=False),
    )(_sc_body)
    he = jnp.asarray(hyperedge_index, jnp.int32)
    return sc_fn(ttx, tte, he, cpad)
